# baseline (speedup 1.0000x reference)
import functools

import numpy as np
import jax
import jax.numpy as jnp
from jax import lax
from jax.experimental import pallas as pl
from jax.experimental.pallas import tpu as pltpu
from jax.experimental.pallas import tpu_sc as plsc

_NC = 2
_NS = 16
_L = 16
_NW = _NC * _NS

_CHUNK = 16384
_NBUF = 4

_B1 = jnp.float32(np.uint32(1).view(np.float32))
_B2 = jnp.float32(np.uint32(2).view(np.float32))
_B3 = jnp.float32(np.uint32(3).view(np.float32))


@functools.lru_cache(maxsize=None)
def _make_bucketize(n: int):
    per_w = n // _NW
    n_chunks = per_w // _CHUNK
    assert per_w % _CHUNK == 0 and n % _NW == 0 and n_chunks % _NBUF == 0

    mesh = plsc.VectorSubcoreMesh(
        core_axis_name="c", subcore_axis_name="s",
        num_cores=_NC, num_subcores=_NS)

    @functools.partial(
        pl.kernel,
        out_type=(jax.ShapeDtypeStruct((n,), jnp.float32),
                  jax.ShapeDtypeStruct((n,), jnp.float32)),
        mesh=mesh,
        scratch_types=(
            [pltpu.VMEM((_CHUNK,), jnp.float32)] * _NBUF
            + [pltpu.VMEM((_CHUNK,), jnp.float32)]
            + [pltpu.SemaphoreType.DMA] * (2 * _NBUF + 1)
        ),
    )
    def bucketize(x_hbm, out_hbm, w_hbm, *refs):
        bufs = refs[:_NBUF]
        wv = refs[_NBUF]
        sin = refs[_NBUF + 1:2 * _NBUF + 1]
        sout = refs[2 * _NBUF + 1:3 * _NBUF + 1]
        sw = refs[3 * _NBUF + 1]

        wid = lax.axis_index("s") * _NC + lax.axis_index("c")
        base = wid * per_w

        def in_copy(k, b):
            return pltpu.make_async_copy(
                x_hbm.at[pl.ds(base + k * _CHUNK, _CHUNK)], bufs[b], sin[b])

        def out_copy(k, b):
            return pltpu.make_async_copy(
                bufs[b], out_hbm.at[pl.ds(base + k * _CHUNK, _CHUNK)], sout[b])

        def w_copy(k):
            return pltpu.make_async_copy(
                wv, w_hbm.at[pl.ds(base + k * _CHUNK, _CHUNK)], sw)

        for b in range(_NBUF - 1):
            in_copy(b, b).start()

        @plsc.parallel_loop(0, _CHUNK // _L, unroll=16)
        def _(i):
            wv[pl.ds(i * _L, _L)] = jnp.full((_L,), 1.0, jnp.float32)

        def chunk_body(j, carry):
            for b in range(_NBUF):
                k = j * _NBUF + b
                w_copy(k).start()
                in_copy(k, b).wait()

                @plsc.parallel_loop(0, _CHUNK // _L, unroll=16)
                def _(i):
                    v = bufs[b][pl.ds(i * _L, _L)]
                    bufs[b][pl.ds(i * _L, _L)] = jnp.where(
                        v >= 0.0,
                        jnp.where(v >= 2.0,
                                  jnp.where(v >= 4.0, _B3, _B2),
                                  _B1),
                        jnp.float32(0.0))

                out_copy(k, b).start()

                bp = (b + _NBUF - 1) % _NBUF

                @pl.when(k >= 1)
                def _():
                    out_copy(k - 1, bp).wait()

                @pl.when(k + _NBUF - 1 < n_chunks)
                def _():
                    in_copy(k + _NBUF - 1, bp).start()
            return carry

        lax.fori_loop(0, n_chunks // _NBUF, chunk_body, 0)
        out_copy(n_chunks - 1, (n_chunks - 1) % _NBUF).wait()

        def w_drain(j, carry):
            w_copy(0).wait()
            return carry

        lax.fori_loop(0, n_chunks, w_drain, 0)

    return bucketize


def kernel(inputs):
    x = jnp.asarray(inputs, jnp.float32)
    n = x.shape[0]
    ids_f32, weights = _make_bucketize(n)(x.reshape(n))
    ids = lax.bitcast_convert_type(ids_f32, jnp.int32)
    return (ids.reshape(n, 1), weights.reshape(n, 1))


# speedup vs baseline: 1.4074x; 1.4074x over previous
import functools

import jax
import jax.numpy as jnp
from jax import lax
from jax.experimental import pallas as pl
from jax.experimental.pallas import tpu as pltpu
from jax.experimental.pallas import tpu_sc as plsc

_NC = 2
_NS = 16
_L = 16
_NW = _NC * _NS

_CHUNK = 16384
_NIN = 4
_NOUT = 2


@functools.lru_cache(maxsize=None)
def _make_bucketize(n: int):
    per_w = n // _NW
    n_chunks = per_w // _CHUNK
    assert per_w % _CHUNK == 0 and n % _NW == 0 and n_chunks % _NIN == 0

    mesh = plsc.VectorSubcoreMesh(
        core_axis_name="c", subcore_axis_name="s",
        num_cores=_NC, num_subcores=_NS)

    @functools.partial(
        pl.kernel,
        out_type=(jax.ShapeDtypeStruct((n,), jnp.int32),
                  jax.ShapeDtypeStruct((n,), jnp.float32)),
        mesh=mesh,
        scratch_types=(
            [pltpu.VMEM((_CHUNK,), jnp.float32)] * _NIN
            + [pltpu.VMEM((_CHUNK,), jnp.int32)] * _NOUT
            + [pltpu.VMEM((_CHUNK,), jnp.float32)]
            + [pltpu.SemaphoreType.DMA] * (_NIN + _NOUT + 1)
        ),
    )
    def bucketize(x_hbm, out_hbm, w_hbm, *refs):
        xvs = refs[:_NIN]
        bvs = refs[_NIN:_NIN + _NOUT]
        wv = refs[_NIN + _NOUT]
        sin = refs[_NIN + _NOUT + 1:2 * _NIN + _NOUT + 1]
        sout = refs[2 * _NIN + _NOUT + 1:2 * _NIN + 2 * _NOUT + 1]
        sw = refs[2 * _NIN + 2 * _NOUT + 1]

        wid = lax.axis_index("s") * _NC + lax.axis_index("c")
        base = wid * per_w

        def in_copy(k, b):
            return pltpu.make_async_copy(
                x_hbm.at[pl.ds(base + k * _CHUNK, _CHUNK)], xvs[b], sin[b])

        def out_copy(k, b):
            return pltpu.make_async_copy(
                bvs[b], out_hbm.at[pl.ds(base + k * _CHUNK, _CHUNK)], sout[b])

        def w_copy(k):
            return pltpu.make_async_copy(
                wv, w_hbm.at[pl.ds(base + k * _CHUNK, _CHUNK)], sw)

        for b in range(_NIN):
            in_copy(b, b).start()

        @plsc.parallel_loop(0, _CHUNK // _L, unroll=16)
        def _(i):
            wv[pl.ds(i * _L, _L)] = jnp.full((_L,), 1.0, jnp.float32)

        def chunk_body(j, carry):
            for ib in range(_NIN):
                k = j * _NIN + ib
                ob = ib % _NOUT
                w_copy(k).start()
                in_copy(k, ib).wait()

                @pl.when(k >= _NOUT)
                def _():
                    out_copy(k, ob).wait()

                @plsc.parallel_loop(0, _CHUNK // _L, unroll=16)
                def _(i):
                    v = xvs[ib][pl.ds(i * _L, _L)]
                    bvs[ob][pl.ds(i * _L, _L)] = jnp.where(
                        v >= 0.0,
                        jnp.where(v >= 2.0, jnp.where(v >= 4.0, 3, 2), 1),
                        0)

                out_copy(k, ob).start()

                @pl.when(k + _NIN < n_chunks)
                def _():
                    in_copy(k + _NIN, ib).start()
            return carry

        lax.fori_loop(0, n_chunks // _NIN, chunk_body, 0)
        for k in range(n_chunks - _NOUT, n_chunks):
            out_copy(k, k % _NOUT).wait()

        def w_drain(j, carry):
            w_copy(0).wait()
            return carry

        lax.fori_loop(0, n_chunks, w_drain, 0)

    return bucketize


def kernel(inputs):
    x = jnp.asarray(inputs, jnp.float32)
    n = x.shape[0]
    ids, weights = _make_bucketize(n)(x.reshape(n))
    return (ids.reshape(n, 1), weights.reshape(n, 1))


# speedup vs baseline: 1.4098x; 1.0017x over previous
import functools

import jax
import jax.numpy as jnp
from jax import lax
from jax.experimental import pallas as pl
from jax.experimental.pallas import tpu as pltpu
from jax.experimental.pallas import tpu_sc as plsc

_NC = 2
_NS = 16
_L = 16
_NW = _NC * _NS

_CHUNK = 16384
_NIN = 4
_NOUT = 2


@functools.lru_cache(maxsize=None)
def _make_bucketize(n: int):
    per_w = n // _NW
    n_chunks = per_w // _CHUNK
    assert per_w % _CHUNK == 0 and n % _NW == 0 and n_chunks % _NIN == 0

    mesh = plsc.VectorSubcoreMesh(
        core_axis_name="c", subcore_axis_name="s",
        num_cores=_NC, num_subcores=_NS)

    @functools.partial(
        pl.kernel,
        out_type=(jax.ShapeDtypeStruct((n,), jnp.int32),
                  jax.ShapeDtypeStruct((n,), jnp.float32)),
        mesh=mesh,
        scratch_types=(
            [pltpu.VMEM((_CHUNK,), jnp.float32)] * _NIN
            + [pltpu.VMEM((_CHUNK,), jnp.int32)] * _NOUT
            + [pltpu.VMEM((_CHUNK,), jnp.float32)]
            + [pltpu.SemaphoreType.DMA] * (_NIN + _NOUT + 1)
        ),
    )
    def bucketize(x_hbm, out_hbm, w_hbm, *refs):
        xvs = refs[:_NIN]
        bvs = refs[_NIN:_NIN + _NOUT]
        wv = refs[_NIN + _NOUT]
        sin = refs[_NIN + _NOUT + 1:2 * _NIN + _NOUT + 1]
        sout = refs[2 * _NIN + _NOUT + 1:2 * _NIN + 2 * _NOUT + 1]
        sw = refs[2 * _NIN + 2 * _NOUT + 1]

        wid = lax.axis_index("s") * _NC + lax.axis_index("c")
        base = wid * per_w

        def in_copy(k, b):
            return pltpu.make_async_copy(
                x_hbm.at[pl.ds(base + k * _CHUNK, _CHUNK)], xvs[b], sin[b])

        def out_copy(k, b):
            return pltpu.make_async_copy(
                bvs[b], out_hbm.at[pl.ds(base + k * _CHUNK, _CHUNK)], sout[b])

        def w_copy(k):
            return pltpu.make_async_copy(
                wv, w_hbm.at[pl.ds(base + k * _CHUNK, _CHUNK)], sw)

        for b in range(_NIN):
            in_copy(b, b).start()

        @plsc.parallel_loop(0, _CHUNK // _L, unroll=8)
        def _(i):
            wv[pl.ds(i * _L, _L)] = jnp.full((_L,), 1.0, jnp.float32)

        def chunk_body(j, carry):
            for ib in range(_NIN):
                k = j * _NIN + ib
                ob = ib % _NOUT
                w_copy(k).start()
                in_copy(k, ib).wait()

                @pl.when(k >= _NOUT)
                def _():
                    out_copy(k, ob).wait()

                @plsc.parallel_loop(0, _CHUNK // _L, unroll=8)
                def _(i):
                    v = xvs[ib][pl.ds(i * _L, _L)]
                    bvs[ob][pl.ds(i * _L, _L)] = jnp.where(
                        v >= 0.0,
                        jnp.where(v >= 2.0, jnp.where(v >= 4.0, 3, 2), 1),
                        0)

                out_copy(k, ob).start()

                @pl.when(k + _NIN < n_chunks)
                def _():
                    in_copy(k + _NIN, ib).start()
            return carry

        lax.fori_loop(0, n_chunks // _NIN, chunk_body, 0)
        for k in range(n_chunks - _NOUT, n_chunks):
            out_copy(k, k % _NOUT).wait()

        def w_drain(j, carry):
            w_copy(0).wait()
            return carry

        lax.fori_loop(0, n_chunks, w_drain, 0)

    return bucketize


def kernel(inputs):
    x = jnp.asarray(inputs, jnp.float32)
    n = x.shape[0]
    ids, weights = _make_bucketize(n)(x.reshape(n))
    return (ids.reshape(n, 1), weights.reshape(n, 1))


# speedup vs baseline: 1.4609x; 1.0363x over previous
import functools

import jax
import jax.numpy as jnp
from jax import lax
from jax.experimental import pallas as pl
from jax.experimental.pallas import tpu as pltpu
from jax.experimental.pallas import tpu_sc as plsc

_NC = 2
_NS = 16
_L = 16
_NW = _NC * _NS

_CHUNK = 16384
_NIN = 4
_NOUT = 2


@functools.lru_cache(maxsize=None)
def _make_bucketize(n: int):
    per_w = n // _NW
    n_chunks = per_w // _CHUNK
    assert per_w % _CHUNK == 0 and n % _NW == 0 and n_chunks % _NIN == 0
    assert per_w == _NS * _CHUNK

    mesh = plsc.VectorSubcoreMesh(
        core_axis_name="c", subcore_axis_name="s",
        num_cores=_NC, num_subcores=_NS)

    @functools.partial(
        pl.kernel,
        out_type=(jax.ShapeDtypeStruct((n,), jnp.int32),
                  jax.ShapeDtypeStruct((n,), jnp.float32)),
        mesh=mesh,
        scratch_types=(
            [pltpu.VMEM((_CHUNK,), jnp.float32)] * _NIN
            + [pltpu.VMEM((_CHUNK,), jnp.int32)] * _NOUT
            + [pltpu.VMEM((_CHUNK,), jnp.float32)]
            + [pltpu.VMEM_SHARED((per_w,), jnp.float32)]
            + [pltpu.SemaphoreType.DMA] * (_NIN + _NOUT + 2)
        ),
    )
    def bucketize(x_hbm, out_hbm, w_hbm, *refs):
        xvs = refs[:_NIN]
        bvs = refs[_NIN:_NIN + _NOUT]
        wv = refs[_NIN + _NOUT]
        wshared = refs[_NIN + _NOUT + 1]
        sems = refs[_NIN + _NOUT + 2:]
        sin = sems[:_NIN]
        sout = sems[_NIN:_NIN + _NOUT]
        sw = sems[_NIN + _NOUT]
        swf = sems[_NIN + _NOUT + 1]

        cid = lax.axis_index("c")
        sid = lax.axis_index("s")
        wid = sid * _NC + cid
        base = wid * per_w

        def in_copy(k, b):
            return pltpu.make_async_copy(
                x_hbm.at[pl.ds(base + k * _CHUNK, _CHUNK)], xvs[b], sin[b])

        def out_copy(k, b):
            return pltpu.make_async_copy(
                bvs[b], out_hbm.at[pl.ds(base + k * _CHUNK, _CHUNK)], sout[b])

        for b in range(_NIN):
            in_copy(b, b).start()

        @plsc.parallel_loop(0, _CHUNK // _L, unroll=8)
        def _(i):
            wv[pl.ds(i * _L, _L)] = jnp.full((_L,), 1.0, jnp.float32)

        pltpu.make_async_copy(
            wv, wshared.at[pl.ds(sid * _CHUNK, _CHUNK)], swf).start()
        pltpu.make_async_copy(
            wv, wshared.at[pl.ds(sid * _CHUNK, _CHUNK)], swf).wait()
        plsc.subcore_barrier()
        w_dma = pltpu.make_async_copy(
            wshared, w_hbm.at[pl.ds(base, per_w)], sw)
        w_dma.start()

        def chunk_body(j, carry):
            for ib in range(_NIN):
                k = j * _NIN + ib
                ob = ib % _NOUT
                in_copy(k, ib).wait()

                @pl.when(k >= _NOUT)
                def _():
                    out_copy(k, ob).wait()

                @plsc.parallel_loop(0, _CHUNK // _L, unroll=8)
                def _(i):
                    v = xvs[ib][pl.ds(i * _L, _L)]
                    bvs[ob][pl.ds(i * _L, _L)] = jnp.where(
                        v >= 0.0,
                        jnp.where(v >= 2.0, jnp.where(v >= 4.0, 3, 2), 1),
                        0)

                out_copy(k, ob).start()

                @pl.when(k + _NIN < n_chunks)
                def _():
                    in_copy(k + _NIN, ib).start()
            return carry

        lax.fori_loop(0, n_chunks // _NIN, chunk_body, 0)
        for k in range(n_chunks - _NOUT, n_chunks):
            out_copy(k, k % _NOUT).wait()
        w_dma.wait()

    return bucketize


def kernel(inputs):
    x = jnp.asarray(inputs, jnp.float32)
    n = x.shape[0]
    ids, weights = _make_bucketize(n)(x.reshape(n))
    return (ids.reshape(n, 1), weights.reshape(n, 1))
